# baseline (speedup 1.0000x reference)
import jax
import jax.numpy as jnp
from jax.experimental import pallas as pl
from jax.experimental.pallas import tpu as pltpu

N, C, H, W = 16, 512, 32, 32
ROWS = N * C
COLS = H * W
BR = 256


def _mlp_body(mask_ref, w1_ref, b1_ref, w2_ref, b2_ref, wfc_ref, bfc_ref,
              m_ref, s_ref):
    h = jax.nn.relu(
        jnp.dot(mask_ref[...], w1_ref[...], preferred_element_type=jnp.float32)
        + b1_ref[...])
    m = jax.nn.sigmoid(
        jnp.dot(h, w2_ref[...], preferred_element_type=jnp.float32)
        + b2_ref[...])
    s = jax.nn.sigmoid(
        jnp.dot(m, wfc_ref[...], preferred_element_type=jnp.float32)
        + bfc_ref[...])
    m_ref[...] = m
    s_ref[...] = s


def _exchange_body(sel_ref, lst_ref, gui_ref, out_lst_ref, out_gui_ref):
    cond = sel_ref[...] > 0.5
    l = lst_ref[...]
    g = gui_ref[...]
    out_lst_ref[...] = jnp.where(cond, g, l)
    out_gui_ref[...] = jnp.where(cond, l, g)


def kernel(lst, gui, mask, W1, b1, W2, b2, Wfc, bfc):
    m, s = pl.pallas_call(
        _mlp_body,
        out_shape=(
            jax.ShapeDtypeStruct((N, C), jnp.float32),
            jax.ShapeDtypeStruct((N, C), jnp.float32),
        ),
    )(mask, W1, b1.reshape(1, C), W2, b2.reshape(1, C),
      Wfc, bfc.reshape(1, C))

    sel = s.reshape(ROWS, 1)
    lst2 = lst.reshape(ROWS, COLS)
    gui2 = gui.reshape(ROWS, COLS)

    out_lst, out_gui = pl.pallas_call(
        _exchange_body,
        grid=(ROWS // BR,),
        in_specs=[
            pl.BlockSpec((BR, 1), lambda i: (i, 0)),
            pl.BlockSpec((BR, COLS), lambda i: (i, 0)),
            pl.BlockSpec((BR, COLS), lambda i: (i, 0)),
        ],
        out_specs=[
            pl.BlockSpec((BR, COLS), lambda i: (i, 0)),
            pl.BlockSpec((BR, COLS), lambda i: (i, 0)),
        ],
        out_shape=(
            jax.ShapeDtypeStruct((ROWS, COLS), jnp.float32),
            jax.ShapeDtypeStruct((ROWS, COLS), jnp.float32),
        ),
    )(sel, lst2, gui2)

    return (out_lst.reshape(N, C, H, W), out_gui.reshape(N, C, H, W), m)
